# baseline (speedup 1.0000x reference)
import jax
import jax.numpy as jnp
from jax.experimental import pallas as pl

_T = 4096
_TO = _T // 2
_S = 49
_C = 128
_R = 7168


def _pool_body(x_ref, o_ref):
    o_ref[...] = jnp.maximum(x_ref[0::2, :], x_ref[1::2, :])


def kernel(x, seq_lens):
    xp = x.transpose(2, 3, 0, 1).reshape(_S * _T, _C)
    y = pl.pallas_call(
        _pool_body,
        grid=(_S * _TO // _R,),
        in_specs=[pl.BlockSpec((2 * _R, _C), lambda i: (i, 0))],
        out_specs=pl.BlockSpec((_R, _C), lambda i: (i, 0)),
        out_shape=jax.ShapeDtypeStruct((_S * _TO, _C), jnp.float32),
    )(xp)
    y = y.reshape(7, 7, _TO, _C).transpose(2, 3, 0, 1)
    return (y, jnp.array([_TO], dtype=jnp.int32))
